# Initial kernel scaffold; baseline (speedup 1.0000x reference)
import jax
import jax.numpy as jnp
from jax.experimental import pallas as pl


def kernel(feat_ids, feat_vals, w_first, emb_v, W1, b1, W2, b2, W3, b3, bias):
    raise NotImplementedError("write your pallas kernel here")


# speedup vs baseline: 1.2260x; 1.2260x over previous
import functools

import jax
import jax.numpy as jnp
import numpy as np
from jax import lax
from jax.experimental import pallas as pl
from jax.experimental.pallas import tpu as pltpu
from jax.experimental.pallas import tpu_sc as plsc

B, F, V, D = 16384, 26, 1000000, 16
N = B * F

NC, NS = 2, 16
NW = NC * NS
PER_W = N // NW
IDXW = 128
CHUNK_ROWS = 8
CHUNK = CHUNK_ROWS * IDXW
N_IDX_ROWS = N // IDXW
ROWS_PER_W = N_IDX_ROWS // NW
N_CHUNKS = ROWS_PER_W // CHUNK_ROWS


def _sc_gather(emb_v, w16, idx2):
    mesh = plsc.VectorSubcoreMesh(core_axis_name="c", subcore_axis_name="s")

    @functools.partial(
        pl.kernel,
        mesh=mesh,
        compiler_params=pltpu.CompilerParams(
            use_tc_tiling_on_sc=False, needs_layout_passes=False),
        out_type=(
            jax.ShapeDtypeStruct((N, D), jnp.float32),
            jax.ShapeDtypeStruct((N,), jnp.float32),
        ),
        scratch_types=[
            pltpu.VMEM((CHUNK_ROWS, IDXW), jnp.int32),
            pltpu.VMEM((CHUNK_ROWS, IDXW), jnp.int32),
            pltpu.VMEM((CHUNK, D), jnp.float32),
            pltpu.VMEM((CHUNK, 16), jnp.float32),
            pltpu.VMEM((CHUNK,), jnp.float32),
            pltpu.SemaphoreType.DMA,
            pltpu.SemaphoreType.DMA,
        ],
    )
    def k(emb_hbm, w_hbm, idx_hbm, e_out, w_out,
          idx_v, hi_v, rows_v, wrow_v, wv_v, sem_e, sem_w):
        wid = lax.axis_index("s") * NC + lax.axis_index("c")
        row_base = wid * ROWS_PER_W
        iota16 = lax.iota(jnp.int32, 16)

        @pl.loop(0, N_CHUNKS)
        def _(c):
            r0 = row_base + c * CHUNK_ROWS
            pltpu.sync_copy(idx_hbm.at[pl.ds(r0, CHUNK_ROWS)], idx_v)

            @pl.loop(0, CHUNK_ROWS)
            def _(r):
                for g in range(IDXW // 16):
                    seg = idx_v[r, pl.ds(g * 16, 16)]
                    hi_v[r, pl.ds(g * 16, 16)] = jnp.right_shift(seg, 4)

            cps = []
            for j in range(CHUNK_ROWS):
                cps.append(pltpu.async_copy(
                    emb_hbm.at[idx_v.at[j]],
                    rows_v.at[pl.ds(j * IDXW, IDXW)], sem_e))
                cps.append(pltpu.async_copy(
                    w_hbm.at[hi_v.at[j]],
                    wrow_v.at[pl.ds(j * IDXW, IDXW)], sem_w))
            for cp in cps:
                cp.wait()

            @pl.loop(0, CHUNK_ROWS)
            def _(r):
                for g in range(IDXW // 16):
                    pos = r * IDXW + g * 16
                    seg = idx_v[r, pl.ds(g * 16, 16)]
                    lo = jnp.bitwise_and(seg, 15)
                    vals = plsc.load_gather(wrow_v, [iota16 + pos, lo])
                    wv_v[pl.ds(pos, 16)] = vals

            base = r0 * IDXW
            pltpu.sync_copy(rows_v, e_out.at[pl.ds(base, CHUNK)])
            pltpu.sync_copy(wv_v, w_out.at[pl.ds(base, CHUNK)])

    return k(emb_v, w16, idx2)


BBLK = 1024


def _fm_mlp_body(e_ref, vals_ref, wf_ref, W1_ref, b1_ref, W2_ref, b2_ref,
                 W3_ref, b3f_ref, R_ref, S_ref, out_ref):
    vals = vals_ref[...]
    vrep = jnp.dot(vals, R_ref[...], preferred_element_type=jnp.float32)
    ev = e_ref[...] * vrep
    first = jnp.sum(wf_ref[...] * vals, axis=1)
    S = S_ref[...]
    sum_e = jnp.dot(ev, S, preferred_element_type=jnp.float32)
    sum_sq = jnp.dot(ev * ev, S, preferred_element_type=jnp.float32)
    second = 0.5 * jnp.sum(sum_e * sum_e - sum_sq, axis=1)
    h = jnp.maximum(jnp.dot(ev, W1_ref[...], preferred_element_type=jnp.float32)
                    + b1_ref[...], 0.0)
    h = jnp.maximum(jnp.dot(h, W2_ref[...], preferred_element_type=jnp.float32)
                    + b2_ref[...], 0.0)
    deep = jnp.dot(h, W3_ref[...], preferred_element_type=jnp.float32)[:, 0]
    logits = first + second + deep + b3f_ref[0, 0]
    out_ref[...] = 1.0 / (1.0 + jnp.exp(-logits))


def _fm_mlp(e2, vals, wf, W1, b1, W2, b2, W3, b3f, R, S):
    grid = (B // BBLK,)
    return pl.pallas_call(
        _fm_mlp_body,
        grid=grid,
        in_specs=[
            pl.BlockSpec((BBLK, F * D), lambda i: (i, 0)),
            pl.BlockSpec((BBLK, F), lambda i: (i, 0)),
            pl.BlockSpec((BBLK, F), lambda i: (i, 0)),
            pl.BlockSpec((F * D, 256), lambda i: (0, 0)),
            pl.BlockSpec((1, 256), lambda i: (0, 0)),
            pl.BlockSpec((256, 128), lambda i: (0, 0)),
            pl.BlockSpec((1, 128), lambda i: (0, 0)),
            pl.BlockSpec((128, 1), lambda i: (0, 0)),
            pl.BlockSpec((1, 1), lambda i: (0, 0)),
            pl.BlockSpec((F, F * D), lambda i: (0, 0)),
            pl.BlockSpec((F * D, D), lambda i: (0, 0)),
        ],
        out_specs=pl.BlockSpec((BBLK,), lambda i: (i,)),
        out_shape=jax.ShapeDtypeStruct((B,), jnp.float32),
    )(e2, vals, wf, W1, b1, W2, b2, W3, b3f, R, S)


_R_np = np.zeros((F, F * D), dtype=np.float32)
for _f in range(F):
    _R_np[_f, _f * D:(_f + 1) * D] = 1.0
_S_np = np.zeros((F * D, D), dtype=np.float32)
for _f in range(F):
    _S_np[np.arange(_f * D, (_f + 1) * D), np.arange(D)] = 1.0


def kernel(feat_ids, feat_vals, w_first, emb_v, W1, b1, W2, b2, W3, b3, bias):
    idx2 = feat_ids.reshape(N_IDX_ROWS, IDXW)
    w16 = w_first.reshape(V // 16, 16)
    e_raw, wf_flat = _sc_gather(emb_v, w16, idx2)
    e2 = e_raw.reshape(B, F * D)
    wf = wf_flat.reshape(B, F)
    b3f = (b3 + bias).reshape(1, 1)
    R = jnp.asarray(_R_np)
    S = jnp.asarray(_S_np)
    return _fm_mlp(e2, feat_vals, wf, W1, b1.reshape(1, 256), W2,
                   b2.reshape(1, 128), W3, b3f, R, S)
